# Initial kernel scaffold; baseline (speedup 1.0000x reference)
import jax
import jax.numpy as jnp
from jax.experimental import pallas as pl


def kernel(atom_x, voro_x, atom_edge_index, voro_edge_index, batch, atom_in_W, voro_in_W, atom_weight, atom_w_ih, atom_w_hh, atom_b_ih, atom_b_hh, voro_weight, voro_w_ih, voro_w_hh, voro_b_ih, voro_b_hh, pred_W, pred_b):
    raise NotImplementedError("write your pallas kernel here")


# speedup vs baseline: 5.0083x; 5.0083x over previous
import functools

import jax
import jax.numpy as jnp
from jax import lax
from jax.experimental import pallas as pl
from jax.experimental.pallas import tpu as pltpu, tpu_sc as plsc

N_NODES = 10000
N_EDGES = 320000
H = 128
STEPS = 3
NUM_GRAPHS = 64

NC, NS = 2, 16
NW = NC * NS
CH = 128
NCH = 79
EPT = NCH * CH
E_PAD = NW * EPT
ACC_ROWS = 10112
ROWS_PT = ACC_ROWS // NS


def _seg_sum_sc(m, src3, dst3, zrows):
    mesh = plsc.VectorSubcoreMesh(
        core_axis_name="c", subcore_axis_name="s", num_cores=NC, num_subcores=NS
    )

    @functools.partial(
        pl.kernel,
        out_type=jax.ShapeDtypeStruct((NC, ACC_ROWS, H), jnp.float32),
        mesh=mesh,
        scratch_types=[
            pltpu.VMEM((NCH, CH), jnp.int32),
            pltpu.VMEM((NCH, CH), jnp.int32),
            pltpu.VMEM((CH, H), jnp.float32),
            pltpu.VMEM_SHARED((ACC_ROWS, H), jnp.float32),
            pltpu.SemaphoreType.DMA,
        ],
    )
    def k(m_hbm, src_hbm, dst_hbm, z_hbm, out_hbm, src_v, dst_v, rows_v, acc, sem):
        c = lax.axis_index("c")
        s = lax.axis_index("s")
        wid = c * NS + s
        pltpu.sync_copy(
            z_hbm.at[pl.ds(s * ROWS_PT, ROWS_PT)],
            acc.at[pl.ds(s * ROWS_PT, ROWS_PT)],
        )
        pltpu.sync_copy(src_hbm.at[wid], src_v)
        pltpu.sync_copy(dst_hbm.at[wid], dst_v)
        plsc.subcore_barrier()

        def body(g, carry):
            pltpu.async_copy(m_hbm.at[src_v.at[g]], rows_v, sem).wait()
            pltpu.sync_copy(rows_v, acc.at[dst_v.at[g]], add=True)
            return carry

        lax.fori_loop(0, NCH, body, 0)
        plsc.subcore_barrier()
        pltpu.sync_copy(
            acc.at[pl.ds(s * ROWS_PT, ROWS_PT)],
            out_hbm.at[c].at[pl.ds(s * ROWS_PT, ROWS_PT)],
        )

    return k(m, src3, dst3, zrows)


RB = 1000
NRB = N_NODES // RB


def _proj_body(x_ref, wt_ref, w0_ref, h_ref, m_ref):
    h = jnp.tanh(jnp.dot(x_ref[...], wt_ref[...], preferred_element_type=jnp.float32))
    h_ref[...] = h
    m_ref[...] = jnp.dot(h, w0_ref[...], preferred_element_type=jnp.float32)


def _proj_m(x, w_in, w0):
    return pl.pallas_call(
        _proj_body,
        grid=(NRB,),
        in_specs=[
            pl.BlockSpec((RB, H), lambda i: (i, 0)),
            pl.BlockSpec((H, H), lambda i: (0, 0)),
            pl.BlockSpec((H, H), lambda i: (0, 0)),
        ],
        out_specs=[
            pl.BlockSpec((RB, H), lambda i: (i, 0)),
            pl.BlockSpec((RB, H), lambda i: (i, 0)),
        ],
        out_shape=[
            jax.ShapeDtypeStruct((N_NODES, H), jnp.float32),
            jax.ShapeDtypeStruct((N_NODES, H), jnp.float32),
        ],
    )(x, w_in.T, w0)


def _gru_body(p_ref, h_ref, wih_ref, whh_ref, bih_ref, bhh_ref, wn_ref,
              hn_ref, mn_ref):
    agg = p_ref[0] + p_ref[1]
    h = h_ref[...]
    gi = jnp.dot(agg, wih_ref[...], preferred_element_type=jnp.float32) + bih_ref[0:1, :]
    gh = jnp.dot(h, whh_ref[...], preferred_element_type=jnp.float32) + bhh_ref[0:1, :]
    r = jax.nn.sigmoid(gi[:, 0:H] + gh[:, 0:H])
    z = jax.nn.sigmoid(gi[:, H:2 * H] + gh[:, H:2 * H])
    n = jnp.tanh(gi[:, 2 * H:3 * H] + r * gh[:, 2 * H:3 * H])
    hn = (1.0 - z) * n + z * h
    hn_ref[...] = hn
    mn_ref[...] = jnp.dot(hn, wn_ref[...], preferred_element_type=jnp.float32)


def _gru_step(p, h, wih_t, whh_t, bih8, bhh8, w_next):
    return pl.pallas_call(
        _gru_body,
        grid=(NRB,),
        in_specs=[
            pl.BlockSpec((NC, RB, H), lambda i: (0, i, 0)),
            pl.BlockSpec((RB, H), lambda i: (i, 0)),
            pl.BlockSpec((H, 3 * H), lambda i: (0, 0)),
            pl.BlockSpec((H, 3 * H), lambda i: (0, 0)),
            pl.BlockSpec((8, 3 * H), lambda i: (0, 0)),
            pl.BlockSpec((8, 3 * H), lambda i: (0, 0)),
            pl.BlockSpec((H, H), lambda i: (0, 0)),
        ],
        out_specs=[
            pl.BlockSpec((RB, H), lambda i: (i, 0)),
            pl.BlockSpec((RB, H), lambda i: (i, 0)),
        ],
        out_shape=[
            jax.ShapeDtypeStruct((N_NODES, H), jnp.float32),
            jax.ShapeDtypeStruct((N_NODES, H), jnp.float32),
        ],
    )(p, h, wih_t, whh_t, bih8, bhh8, w_next)


def _readout_body(ax_ref, vx_ref, oh_ref, pwa_ref, pwv_ref, out_ref):
    oh = oh_ref[...]
    cnt = jnp.sum(oh, axis=1)
    inv = 1.0 / jnp.maximum(cnt, 1.0)
    ap = jax.nn.relu(jnp.dot(oh, ax_ref[...], preferred_element_type=jnp.float32) * inv[:, None])
    vp = jax.nn.relu(jnp.dot(oh, vx_ref[...], preferred_element_type=jnp.float32) * inv[:, None])
    out_ref[...] = (jnp.dot(ap, pwa_ref[...], preferred_element_type=jnp.float32)
                    + jnp.dot(vp, pwv_ref[...], preferred_element_type=jnp.float32))


def _readout(ax, vx, oh, pwa_pad, pwv_pad):
    return pl.pallas_call(
        _readout_body,
        out_shape=jax.ShapeDtypeStruct((NUM_GRAPHS, H), jnp.float32),
    )(ax, vx, oh, pwa_pad, pwv_pad)


def _prep_edges(edge_index):
    src = edge_index[0]
    dst = edge_index[1]
    pad = E_PAD - N_EDGES
    pr = jnp.arange(pad, dtype=jnp.int32)
    src = jnp.concatenate([src, pr % N_NODES])
    dst = jnp.concatenate([dst, N_NODES + pr % (ACC_ROWS - N_NODES)])
    return src.reshape(NW, NCH, CH), dst.reshape(NW, NCH, CH)


def kernel(atom_x, voro_x, atom_edge_index, voro_edge_index, batch,
           atom_in_W, voro_in_W,
           atom_weight, atom_w_ih, atom_w_hh, atom_b_ih, atom_b_hh,
           voro_weight, voro_w_ih, voro_w_hh, voro_b_ih, voro_b_hh,
           pred_W, pred_b):
    a_src, a_dst = _prep_edges(atom_edge_index)
    v_src, v_dst = _prep_edges(voro_edge_index)
    zrows = jnp.zeros((ACC_ROWS, H), jnp.float32)

    a_wih_t = atom_w_ih.T
    a_whh_t = atom_w_hh.T
    v_wih_t = voro_w_ih.T
    v_whh_t = voro_w_hh.T
    a_bih8 = jnp.broadcast_to(atom_b_ih[None, :], (8, 3 * H))
    a_bhh8 = jnp.broadcast_to(atom_b_hh[None, :], (8, 3 * H))
    v_bih8 = jnp.broadcast_to(voro_b_ih[None, :], (8, 3 * H))
    v_bhh8 = jnp.broadcast_to(voro_b_hh[None, :], (8, 3 * H))

    ah, am = _proj_m(atom_x, atom_in_W, atom_weight[0])
    vh, vm = _proj_m(voro_x, voro_in_W, voro_weight[0])

    for i in range(STEPS):
        wa_next = atom_weight[(i + 1) % STEPS]
        wv_next = voro_weight[(i + 1) % STEPS]
        ap = _seg_sum_sc(am, a_src, a_dst, zrows)
        vp = _seg_sum_sc(vm, v_src, v_dst, zrows)
        ah, am = _gru_step(ap, ah, a_wih_t, a_whh_t, a_bih8, a_bhh8, wa_next)
        vh, vm = _gru_step(vp, vh, v_wih_t, v_whh_t, v_bih8, v_bhh8, wv_next)

    oh = (batch[None, :] == jnp.arange(NUM_GRAPHS, dtype=jnp.int32)[:, None]).astype(jnp.float32)
    pwa_pad = jnp.zeros((H, H), jnp.float32).at[:, 0].set(pred_W[0, :H])
    pwv_pad = jnp.zeros((H, H), jnp.float32).at[:, 0].set(pred_W[0, H:])
    ro = _readout(ah, vh, oh, pwa_pad, pwv_pad)
    return ro[:, 0] + pred_b[0]
